# baseline (speedup 1.0000x reference)
import jax
import jax.numpy as jnp
from jax.experimental import pallas as pl
from jax.experimental.pallas import tpu as pltpu

_BLK = 512
_STREAMS = 2


def _half(s, xaug, xr, mi):
    adjb = jnp.where(s > 0.0, 1.0, 0.0).astype(jnp.bfloat16)
    z = jnp.dot(adjb, xaug, preferred_element_type=jnp.float32)
    d = xr.shape[1]
    agg = z[:, :d]
    t2 = z[:, d:d + 1]
    deg0 = z[:, d + 1:d + 2]
    sqr = jnp.sum(xr * xr, axis=1, keepdims=True)
    t3 = jnp.sum(xr * agg, axis=1, keepdims=True)
    deg = mi * deg0
    diff = mi * (sqr * deg0 + t2 - 2.0 * t3)
    return jnp.tanh(diff / jnp.maximum(deg, 1.0))


def _g2_kernel(*refs):
    s_refs = refs[:_STREAMS]
    f_ref, w_ref, b_ref, m_ref, out_ref, xs_ref, xa_ref = refs[_STREAMS:]
    i = pl.program_id(1)

    @pl.when(i == 0)
    def _compute_x():
        x = jnp.dot(f_ref[0], w_ref[...], preferred_element_type=jnp.float32)
        x = jnp.maximum(x + b_ref[...], 0.0)
        xs_ref[...] = x
        n, d = x.shape
        sq = jnp.sum(x * x, axis=1, keepdims=True)
        lane = jax.lax.broadcasted_iota(jnp.int32, (n, d), 1)
        extra = jnp.where(lane == 0, sq, jnp.where(lane == 1, 1.0, 0.0))
        xa_ref[...] = (jnp.concatenate([x, extra], axis=1)
                       * m_ref[0]).astype(jnp.bfloat16)

    xaug = xa_ref[...]
    h = _BLK
    for k in range(_STREAMS):
        row0 = (i * _STREAMS + k) * _BLK
        xr = xs_ref[pl.ds(row0, h), :]
        mi = m_ref[0, pl.ds(row0, h), :]
        out_ref[0, k * h:(k + 1) * h] = _half(s_refs[k][0], xaug, xr, mi)


def kernel(features, support, mask, W, b):
    B, N, D = features.shape
    S = _STREAMS
    sup_specs = [
        pl.BlockSpec((1, _BLK, N), lambda bb, i, k=k: (bb, S * i + k, 0))
        for k in range(S)
    ]
    tau = pl.pallas_call(
        _g2_kernel,
        grid=(B, N // (S * _BLK)),
        in_specs=sup_specs + [
            pl.BlockSpec((1, N, D), lambda bb, i: (bb, 0, 0)),
            pl.BlockSpec((D, D), lambda bb, i: (0, 0)),
            pl.BlockSpec((1, D), lambda bb, i: (0, 0)),
            pl.BlockSpec((1, N, 1), lambda bb, i: (bb, 0, 0)),
        ],
        out_specs=pl.BlockSpec((1, S * _BLK, 1), lambda bb, i: (bb, i, 0)),
        out_shape=jax.ShapeDtypeStruct((B, N, 1), jnp.float32),
        scratch_shapes=[
            pltpu.VMEM((N, D), jnp.float32),
            pltpu.VMEM((N, 2 * D), jnp.bfloat16),
        ],
        compiler_params=pltpu.CompilerParams(
            dimension_semantics=("arbitrary", "arbitrary")),
    )(support, *([support] * (S - 1)), features, W, b.reshape(1, D), mask)
    probe = jnp.max(support, axis=2, keepdims=True) * 0.0
    return tau + probe


# speedup vs baseline: 1.8585x; 1.8585x over previous
import jax
import jax.numpy as jnp
from jax.experimental import pallas as pl
from jax.experimental.pallas import tpu as pltpu

_BLK = 512
_DEPTH = 4


def _half(s, xaug, xr, mi):
    adjb = jnp.where(s > 0.0, 1.0, 0.0).astype(jnp.bfloat16)
    z = jnp.dot(adjb, xaug, preferred_element_type=jnp.float32)
    d = xr.shape[1]
    agg = z[:, :d]
    t2 = z[:, d:d + 1]
    deg0 = z[:, d + 1:d + 2]
    sqr = jnp.sum(xr * xr, axis=1, keepdims=True)
    t3 = jnp.sum(xr * agg, axis=1, keepdims=True)
    deg = mi * deg0
    diff = mi * (sqr * deg0 + t2 - 2.0 * t3)
    return jnp.tanh(diff / jnp.maximum(deg, 1.0))


def _g2_kernel(s_hbm, f_ref, w_ref, b_ref, m_ref, out_ref,
               buf_ref, xs_ref, xa_ref, sems):
    B, N, _ = f_ref.shape
    D = f_ref.shape[2]
    nb = N // _BLK
    T = B * nb

    def start(t):
        bb = t // nb
        row0 = (t % nb) * _BLK
        slot = jax.lax.rem(t, _DEPTH)
        pltpu.make_async_copy(
            s_hbm.at[bb, pl.ds(row0, _BLK), :],
            buf_ref.at[slot],
            sems.at[slot],
        ).start()

    for t in range(_DEPTH):
        start(t)

    for bb in range(B):
        x = jnp.dot(f_ref[bb], w_ref[...], preferred_element_type=jnp.float32)
        x = jnp.maximum(x + b_ref[...], 0.0)
        xs_ref[bb] = x
        sq = jnp.sum(x * x, axis=1, keepdims=True)
        lane = jax.lax.broadcasted_iota(jnp.int32, (N, D), 1)
        extra = jnp.where(lane == 0, sq, jnp.where(lane == 1, 1.0, 0.0))
        xa_ref[bb] = (jnp.concatenate([x, extra], axis=1)
                      * m_ref[bb]).astype(jnp.bfloat16)

    def body(t, carry):
        bb = t // nb
        row0 = (t % nb) * _BLK
        slot = jax.lax.rem(t, _DEPTH)
        pltpu.make_async_copy(
            s_hbm.at[bb, pl.ds(row0, _BLK), :],
            buf_ref.at[slot],
            sems.at[slot],
        ).wait()
        s = buf_ref[slot]
        xaug = xa_ref[bb]
        xr = xs_ref[bb, pl.ds(row0, _BLK), :]
        mi = m_ref[bb, pl.ds(row0, _BLK), :]
        out_ref[bb, pl.ds(row0, _BLK), :] = _half(s, xaug, xr, mi)

        @pl.when(t + _DEPTH < T)
        def _():
            start(t + _DEPTH)

        return carry

    jax.lax.fori_loop(0, T, body, 0)


def kernel(features, support, mask, W, b):
    B, N, D = features.shape
    tau = pl.pallas_call(
        _g2_kernel,
        in_specs=[
            pl.BlockSpec(memory_space=pltpu.MemorySpace.HBM),
            pl.BlockSpec(memory_space=pltpu.MemorySpace.VMEM),
            pl.BlockSpec(memory_space=pltpu.MemorySpace.VMEM),
            pl.BlockSpec(memory_space=pltpu.MemorySpace.VMEM),
            pl.BlockSpec(memory_space=pltpu.MemorySpace.VMEM),
        ],
        out_specs=pl.BlockSpec(memory_space=pltpu.MemorySpace.VMEM),
        out_shape=jax.ShapeDtypeStruct((B, N, 1), jnp.float32),
        scratch_shapes=[
            pltpu.VMEM((_DEPTH, _BLK, N), jnp.float32),
            pltpu.VMEM((B, N, D), jnp.float32),
            pltpu.VMEM((B, N, 2 * D), jnp.bfloat16),
            pltpu.SemaphoreType.DMA((_DEPTH,)),
        ],
    )(support, features, W, b.reshape(1, D), mask)
    return tau


# speedup vs baseline: 1.8661x; 1.0041x over previous
import jax
import jax.numpy as jnp
from jax.experimental import pallas as pl
from jax.experimental.pallas import tpu as pltpu

_BLK = 256
_DEPTH = 8


def _half(s, xaug, xr, mi):
    adjb = jnp.where(s > 0.0, 1.0, 0.0).astype(jnp.bfloat16)
    z = jnp.dot(adjb, xaug, preferred_element_type=jnp.float32)
    d = xr.shape[1]
    agg = z[:, :d]
    t2 = z[:, d:d + 1]
    deg0 = z[:, d + 1:d + 2]
    sqr = jnp.sum(xr * xr, axis=1, keepdims=True)
    t3 = jnp.sum(xr * agg, axis=1, keepdims=True)
    deg = mi * deg0
    diff = mi * (sqr * deg0 + t2 - 2.0 * t3)
    return jnp.tanh(diff / jnp.maximum(deg, 1.0))


def _g2_kernel(s_hbm, f_ref, w_ref, b_ref, m_ref, out_ref,
               buf_ref, xs_ref, xa_ref, sems):
    B, N, _ = f_ref.shape
    D = f_ref.shape[2]
    nb = N // _BLK
    T = B * nb

    def start(t):
        bb = t // nb
        row0 = (t % nb) * _BLK
        slot = jax.lax.rem(t, _DEPTH)
        pltpu.make_async_copy(
            s_hbm.at[bb, pl.ds(row0, _BLK), :],
            buf_ref.at[slot],
            sems.at[slot],
        ).start()

    for t in range(_DEPTH):
        start(t)

    for bb in range(B):
        x = jnp.dot(f_ref[bb], w_ref[...], preferred_element_type=jnp.float32)
        x = jnp.maximum(x + b_ref[...], 0.0)
        xs_ref[bb] = x
        sq = jnp.sum(x * x, axis=1, keepdims=True)
        lane = jax.lax.broadcasted_iota(jnp.int32, (N, D), 1)
        extra = jnp.where(lane == 0, sq, jnp.where(lane == 1, 1.0, 0.0))
        xa_ref[bb] = (jnp.concatenate([x, extra], axis=1)
                      * m_ref[bb]).astype(jnp.bfloat16)

    def body(t, carry):
        bb = t // nb
        row0 = (t % nb) * _BLK
        slot = jax.lax.rem(t, _DEPTH)
        pltpu.make_async_copy(
            s_hbm.at[bb, pl.ds(row0, _BLK), :],
            buf_ref.at[slot],
            sems.at[slot],
        ).wait()
        s = buf_ref[slot]
        xaug = xa_ref[bb]
        xr = xs_ref[bb, pl.ds(row0, _BLK), :]
        mi = m_ref[bb, pl.ds(row0, _BLK), :]
        out_ref[bb, pl.ds(row0, _BLK), :] = _half(s, xaug, xr, mi)

        @pl.when(t + _DEPTH < T)
        def _():
            start(t + _DEPTH)

        return carry

    jax.lax.fori_loop(0, T, body, 0)


def kernel(features, support, mask, W, b):
    B, N, D = features.shape
    tau = pl.pallas_call(
        _g2_kernel,
        in_specs=[
            pl.BlockSpec(memory_space=pltpu.MemorySpace.HBM),
            pl.BlockSpec(memory_space=pltpu.MemorySpace.VMEM),
            pl.BlockSpec(memory_space=pltpu.MemorySpace.VMEM),
            pl.BlockSpec(memory_space=pltpu.MemorySpace.VMEM),
            pl.BlockSpec(memory_space=pltpu.MemorySpace.VMEM),
        ],
        out_specs=pl.BlockSpec(memory_space=pltpu.MemorySpace.VMEM),
        out_shape=jax.ShapeDtypeStruct((B, N, 1), jnp.float32),
        scratch_shapes=[
            pltpu.VMEM((_DEPTH, _BLK, N), jnp.float32),
            pltpu.VMEM((B, N, D), jnp.float32),
            pltpu.VMEM((B, N, 2 * D), jnp.bfloat16),
            pltpu.SemaphoreType.DMA((_DEPTH,)),
        ],
    )(support, features, W, b.reshape(1, D), mask)
    return tau
